# baseline (speedup 1.0000x reference)
import functools
import jax
import jax.numpy as jnp
from jax import lax
from jax.experimental import pallas as pl
from jax.experimental.pallas import tpu as pltpu
from jax.experimental.pallas import tpu_sc as plsc

B, N, M = 8, 4096, 1024
C_KNOWN, C_UNKNOWN = 256, 128
BN = B * N
NBLK = 512
ROWBLK = 1024
K = 3

SLICES = 4
BS = B // SLICES
SN = BS * N

NUM_CORES = 2
NUM_SUBCORES = 16
NW = NUM_CORES * NUM_SUBCORES
PTS_PER_W = SN // NW
G = 32
NCHUNK = PTS_PER_W // G
NBUF = 4


def _topk_body(base_b, ut_ref, k_ref, *out_refs):
    b = pl.program_id(0)
    ut = ut_ref[0]
    kk = k_ref[0]
    d = jnp.zeros((M, NBLK), jnp.float32)
    for j in range(3):
        diff = kk[:, j:j + 1] - ut[j:j + 1, :]
        d = d + diff * diff
    iota = lax.broadcasted_iota(jnp.int32, (M, NBLK), 0)
    idx_refs = out_refs[:K]
    w_refs = out_refs[K:]
    ws = []
    for r in range(K):
        m = jnp.min(d, axis=0, keepdims=True)
        cand = jnp.where(d == m, iota, jnp.int32(2 ** 30))
        a = jnp.min(cand, axis=0, keepdims=True)
        ws.append(1.0 / (m + 1e-8))
        idx_refs[r][...] = a[0] + (b + base_b) * M
        if r < K - 1:
            d = jnp.where(iota == a, jnp.float32(1e30), d)
    wsum = ws[0] + ws[1] + ws[2]
    for r in range(K):
        w_refs[r][...] = (ws[r] / wsum)[0]


def _topk(unknown_pc_t_s, known_pc_s, base_b):
    nb = N // NBLK
    return pl.pallas_call(
        functools.partial(_topk_body, base_b),
        grid=(BS, nb),
        in_specs=[
            pl.BlockSpec((1, 3, NBLK), lambda b, j: (b, 0, j)),
            pl.BlockSpec((1, M, 3), lambda b, j: (b, 0, 0)),
        ],
        out_specs=[pl.BlockSpec((NBLK,), lambda b, j: (b * nb + j,))] * (2 * K),
        out_shape=[jax.ShapeDtypeStruct((SN,), jnp.int32)] * K
        + [jax.ShapeDtypeStruct((SN,), jnp.float32)] * K,
    )(unknown_pc_t_s, known_pc_s)


def _interp_body(table_hbm, i0_hbm, i1_hbm, i2_hbm, w0_hbm, w1_hbm, w2_hbm,
                 out_hbm, i0_v, i1_v, i2_v, w0_v, w1_v, w2_v,
                 r0_v, r1_v, r2_v, ob_v, sem_g, sem_o):
    wid = lax.axis_index("s") * NUM_CORES + lax.axis_index("c")
    base_pt = wid * PTS_PER_W
    lane = lax.broadcasted_iota(jnp.int32, (16,), 0)
    i_hbms = [i0_hbm, i1_hbm, i2_hbm]
    w_hbms = [w0_hbm, w1_hbm, w2_hbm]
    idx_vs = [i0_v, i1_v, i2_v]
    w_vs = [w0_v, w1_v, w2_v]
    rows_vs = [r0_v, r1_v, r2_v]

    for k in range(K):
        pltpu.sync_copy(i_hbms[k].at[pl.ds(base_pt, PTS_PER_W)], idx_vs[k])
        pltpu.sync_copy(w_hbms[k].at[pl.ds(base_pt, PTS_PER_W)], w_vs[k])

    def start_gather(g, buf):
        for k in range(K):
            pltpu.async_copy(
                table_hbm.at[idx_vs[k].at[pl.ds(g * G, G)]],
                rows_vs[k].at[buf], sem_g[buf])

    def wait_gather(g, buf):
        for k in range(K):
            pltpu.make_async_copy(
                table_hbm.at[idx_vs[k].at[pl.ds(g * G, G)]],
                rows_vs[k].at[buf], sem_g[buf]).wait()

    def compute(g, buf, obuf):
        def pt_body(p, _):
            woff = g * G + p
            wv = [plsc.load_gather(w_vs[k], [jnp.broadcast_to(woff, (16,))])
                  for k in range(K)]
            psel = jnp.broadcast_to(p, (16,))
            for c in range(C_KNOWN // 16):
                col = c * 16 + lane
                acc = wv[0] * plsc.load_gather(r0_v.at[buf], [psel, col])
                acc = acc + wv[1] * plsc.load_gather(r1_v.at[buf], [psel, col])
                acc = acc + wv[2] * plsc.load_gather(r2_v.at[buf], [psel, col])
                plsc.store_scatter(ob_v.at[obuf], [psel, col], acc)
            return 0

        lax.fori_loop(0, G, pt_body, 0, unroll=False)

    def start_out(g, buf):
        pltpu.async_copy(ob_v.at[buf],
                         out_hbm.at[pl.ds(base_pt + g * G, G)], sem_o[buf])

    def wait_out(g, buf):
        pltpu.make_async_copy(ob_v.at[buf],
                              out_hbm.at[pl.ds(base_pt + g * G, G)],
                              sem_o[buf]).wait()

    for g in range(min(NBUF, NCHUNK)):
        start_gather(g, g % NBUF)
    for g in range(NCHUNK):
        buf = g % NBUF
        obuf = g % 2
        if g > 1:
            wait_out(g - 2, obuf)
        wait_gather(g, buf)
        compute(g, buf, obuf)
        if g + NBUF < NCHUNK:
            start_gather(g + NBUF, buf)
        start_out(g, obuf)
    wait_out(NCHUNK - 2, 0)
    wait_out(NCHUNK - 1, 1)


def _interp_sc(table, idxs, ws):
    mesh = plsc.VectorSubcoreMesh(core_axis_name="c", subcore_axis_name="s")
    fn = pl.kernel(
        _interp_body,
        out_type=jax.ShapeDtypeStruct((SN, C_KNOWN), jnp.float32),
        mesh=mesh,
        compiler_params=pltpu.CompilerParams(needs_layout_passes=False),
        scratch_types=[
            pltpu.VMEM((PTS_PER_W,), jnp.int32),
            pltpu.VMEM((PTS_PER_W,), jnp.int32),
            pltpu.VMEM((PTS_PER_W,), jnp.int32),
            pltpu.VMEM((PTS_PER_W,), jnp.float32),
            pltpu.VMEM((PTS_PER_W,), jnp.float32),
            pltpu.VMEM((PTS_PER_W,), jnp.float32),
            pltpu.VMEM((NBUF, G, C_KNOWN), jnp.float32),
            pltpu.VMEM((NBUF, G, C_KNOWN), jnp.float32),
            pltpu.VMEM((NBUF, G, C_KNOWN), jnp.float32),
            pltpu.VMEM((2, G, C_KNOWN), jnp.float32),
            [pltpu.SemaphoreType.DMA] * NBUF,
            [pltpu.SemaphoreType.DMA, pltpu.SemaphoreType.DMA],
        ],
    )
    return fn(table, idxs[0], idxs[1], idxs[2], ws[0], ws[1], ws[2])


def _mlp1_body(xa_ref, xb_ref, wa_ref, wb_ref, b_ref, y_ref, s_ref, q_ref):
    i = pl.program_id(0)
    y = jnp.dot(xa_ref[...], wa_ref[...], preferred_element_type=jnp.float32)
    y = y + lax.dot_general(xb_ref[0], wb_ref[...],
                            (((0,), (0,)), ((), ())),
                            preferred_element_type=jnp.float32)
    y = y + b_ref[...]
    y_ref[...] = y

    @pl.when(i == 0)
    def _():
        s_ref[...] = jnp.zeros_like(s_ref)
        q_ref[...] = jnp.zeros_like(q_ref)

    s_ref[...] += jnp.sum(y, axis=0, keepdims=True)
    q_ref[...] += jnp.sum(y * y, axis=0, keepdims=True)


def _mlp1(interp, uf_full, base_b, W1aT, W1bT, b1):
    nb = SN // ROWBLK
    nbn = N // ROWBLK
    return pl.pallas_call(
        _mlp1_body,
        grid=(nb,),
        in_specs=[
            pl.BlockSpec((ROWBLK, C_KNOWN), lambda i: (i, 0)),
            pl.BlockSpec((1, C_UNKNOWN, ROWBLK),
                         lambda i: (base_b + i // nbn, 0, i % nbn)),
            pl.BlockSpec((C_KNOWN, 256), lambda i: (0, 0)),
            pl.BlockSpec((C_UNKNOWN, 256), lambda i: (0, 0)),
            pl.BlockSpec((1, 256), lambda i: (0, 0)),
        ],
        out_specs=[
            pl.BlockSpec((ROWBLK, 256), lambda i: (i, 0)),
            pl.BlockSpec((1, 256), lambda i: (0, 0)),
            pl.BlockSpec((1, 256), lambda i: (0, 0)),
        ],
        out_shape=[
            jax.ShapeDtypeStruct((SN, 256), jnp.float32),
            jax.ShapeDtypeStruct((1, 256), jnp.float32),
            jax.ShapeDtypeStruct((1, 256), jnp.float32),
        ],
    )(interp, uf_full, W1aT, W1bT, b1)


def _mlp2_body(x_ref, s1_ref, t1_ref, w_ref, b_ref, y_ref, s_ref, q_ref):
    i = pl.program_id(0)
    h = jnp.maximum(x_ref[...] * s1_ref[...] + t1_ref[...], 0.0)
    y = jnp.dot(h, w_ref[...], preferred_element_type=jnp.float32) + b_ref[...]
    y_ref[...] = y

    @pl.when(i == 0)
    def _():
        s_ref[...] = jnp.zeros_like(s_ref)
        q_ref[...] = jnp.zeros_like(q_ref)

    s_ref[...] += jnp.sum(y, axis=0, keepdims=True)
    q_ref[...] += jnp.sum(y * y, axis=0, keepdims=True)


def _mlp2(y1, s1, t1, W2T, b2):
    nb = SN // ROWBLK
    return pl.pallas_call(
        _mlp2_body,
        grid=(nb,),
        in_specs=[
            pl.BlockSpec((ROWBLK, 256), lambda i: (i, 0)),
            pl.BlockSpec((1, 256), lambda i: (0, 0)),
            pl.BlockSpec((1, 256), lambda i: (0, 0)),
            pl.BlockSpec((256, 256), lambda i: (0, 0)),
            pl.BlockSpec((1, 256), lambda i: (0, 0)),
        ],
        out_specs=[
            pl.BlockSpec((ROWBLK, 256), lambda i: (i, 0)),
            pl.BlockSpec((1, 256), lambda i: (0, 0)),
            pl.BlockSpec((1, 256), lambda i: (0, 0)),
        ],
        out_shape=[
            jax.ShapeDtypeStruct((SN, 256), jnp.float32),
            jax.ShapeDtypeStruct((1, 256), jnp.float32),
            jax.ShapeDtypeStruct((1, 256), jnp.float32),
        ],
    )(y1, s1, t1, W2T, b2)


def _final_body(y0_ref, y1_ref, y2_ref, y3_ref, s2_ref, t2_ref, out_ref):
    i = pl.program_id(0)
    y_refs = [y0_ref, y1_ref, y2_ref, y3_ref]
    nbs = SN // ROWBLK
    for s in range(SLICES):
        @pl.when(i // nbs == s)
        def _():
            h = jnp.maximum(y_refs[s][...] * s2_ref[...] + t2_ref[...], 0.0)
            out_ref[0] = h.T


def _finalize(y2s, s2, t2):
    nbs = SN // ROWBLK
    nbn = N // ROWBLK
    nb = BN // ROWBLK

    def _pinned(s):
        return pl.BlockSpec(
            (ROWBLK, 256),
            lambda i, s=s: (jnp.clip(i - s * nbs, 0, nbs - 1), 0))

    return pl.pallas_call(
        _final_body,
        grid=(nb,),
        in_specs=[_pinned(s) for s in range(SLICES)] + [
            pl.BlockSpec((1, 256), lambda i: (0, 0)),
            pl.BlockSpec((1, 256), lambda i: (0, 0)),
        ],
        out_specs=pl.BlockSpec((1, 256, ROWBLK),
                               lambda i: (i // nbn, 0, i % nbn)),
        out_shape=jax.ShapeDtypeStruct((B, 256, N), jnp.float32),
    )(*y2s, s2, t2)


def _bn_coeffs(ssum, ssq, gamma, beta):
    mean = ssum[0] / BN
    var = ssq[0] / BN - mean * mean
    s = gamma * lax.rsqrt(var + 1e-5)
    t = beta - mean * s
    return s[None, :], t[None, :]


@jax.jit
def kernel(unknown_pc, known_pc, unknow_features, known_features,
           W1, b1, gamma1, beta1, W2, b2, gamma2, beta2):
    uT = jnp.transpose(unknown_pc, (0, 2, 1))
    table = jnp.transpose(known_features, (0, 2, 1)).reshape(B * M, C_KNOWN)

    W1aT = W1[:, :C_KNOWN].T
    W1bT = W1[:, C_KNOWN:].T
    W2T = W2.T
    b1r = b1[None, :]
    b2r = b2[None, :]

    interps = []
    for s in range(SLICES):
        b0 = s * BS
        outs = _topk(uT[b0:b0 + BS], known_pc[b0:b0 + BS], b0)
        interps.append(_interp_sc(table, outs[:K], outs[K:]))

    y1s, s1sums, s1sqs = [], [], []
    for s in range(SLICES):
        y1, ssum, ssq = _mlp1(interps[s], unknow_features, s * BS,
                              W1aT, W1bT, b1r)
        y1s.append(y1)
        s1sums.append(ssum)
        s1sqs.append(ssq)

    s1, t1 = _bn_coeffs(sum(s1sums), sum(s1sqs), gamma1, beta1)

    y2s, s2sums, s2sqs = [], [], []
    for s in range(SLICES):
        y2, ssum, ssq = _mlp2(y1s[s], s1, t1, W2T, b2r)
        y2s.append(y2)
        s2sums.append(ssum)
        s2sqs.append(ssq)

    s2, t2 = _bn_coeffs(sum(s2sums), sum(s2sqs), gamma2, beta2)
    return _finalize(y2s, s2, t2)


# speedup vs baseline: 1.0216x; 1.0216x over previous
import functools
import jax
import jax.numpy as jnp
from jax import lax
from jax.experimental import pallas as pl
from jax.experimental.pallas import tpu as pltpu
from jax.experimental.pallas import tpu_sc as plsc

B, N, M = 8, 4096, 1024
C_KNOWN, C_UNKNOWN = 256, 128
BN = B * N
NBLK = 1024
ROWBLK = 1024
K = 3

SLICES = 4
BS = B // SLICES
SN = BS * N

NUM_CORES = 2
NUM_SUBCORES = 16
NW = NUM_CORES * NUM_SUBCORES
PTS_PER_W = SN // NW
G = 32
NCHUNK = PTS_PER_W // G


def _topk_body(base_b, ut_ref, k_ref, *out_refs):
    b = pl.program_id(0)
    ut = ut_ref[0]
    kk = k_ref[0]
    d = jnp.zeros((M, NBLK), jnp.float32)
    for j in range(3):
        diff = kk[:, j:j + 1] - ut[j:j + 1, :]
        d = d + diff * diff
    iota = lax.broadcasted_iota(jnp.int32, (M, NBLK), 0)
    idx_refs = out_refs[:K]
    w_refs = out_refs[K:]
    ws = []
    for r in range(K):
        m = jnp.min(d, axis=0, keepdims=True)
        cand = jnp.where(d == m, iota, jnp.int32(2 ** 30))
        a = jnp.min(cand, axis=0, keepdims=True)
        ws.append(1.0 / (m + 1e-8))
        idx_refs[r][...] = a[0] + (b + base_b) * M
        if r < K - 1:
            d = jnp.where(iota == a, jnp.float32(1e30), d)
    wsum = ws[0] + ws[1] + ws[2]
    for r in range(K):
        w_refs[r][...] = (ws[r] / wsum)[0]


def _topk(unknown_pc_t_s, known_pc_s, base_b):
    nb = N // NBLK
    return pl.pallas_call(
        functools.partial(_topk_body, base_b),
        grid=(BS, nb),
        in_specs=[
            pl.BlockSpec((1, 3, NBLK), lambda b, j: (b, 0, j)),
            pl.BlockSpec((1, M, 3), lambda b, j: (b, 0, 0)),
        ],
        out_specs=[pl.BlockSpec((NBLK,), lambda b, j: (b * nb + j,))] * (2 * K),
        out_shape=[jax.ShapeDtypeStruct((SN,), jnp.int32)] * K
        + [jax.ShapeDtypeStruct((SN,), jnp.float32)] * K,
    )(unknown_pc_t_s, known_pc_s)


def _interp_body(table_hbm, i0_hbm, i1_hbm, i2_hbm, w0_hbm, w1_hbm, w2_hbm,
                 out_hbm, i0_v, i1_v, i2_v, w0_v, w1_v, w2_v,
                 r0_v, r1_v, r2_v, ob_v, sem_g, sem_o):
    wid = lax.axis_index("s") * NUM_CORES + lax.axis_index("c")
    base_pt = wid * PTS_PER_W
    lane = lax.broadcasted_iota(jnp.int32, (16,), 0)
    i_hbms = [i0_hbm, i1_hbm, i2_hbm]
    w_hbms = [w0_hbm, w1_hbm, w2_hbm]
    idx_vs = [i0_v, i1_v, i2_v]
    w_vs = [w0_v, w1_v, w2_v]
    rows_vs = [r0_v, r1_v, r2_v]

    for k in range(K):
        pltpu.sync_copy(i_hbms[k].at[pl.ds(base_pt, PTS_PER_W)], idx_vs[k])
        pltpu.sync_copy(w_hbms[k].at[pl.ds(base_pt, PTS_PER_W)], w_vs[k])

    def start_gather(g, buf):
        for k in range(K):
            pltpu.async_copy(
                table_hbm.at[idx_vs[k].at[pl.ds(g * G, G)]],
                rows_vs[k].at[buf], sem_g[buf])

    def wait_gather(g, buf):
        for k in range(K):
            pltpu.make_async_copy(
                table_hbm.at[idx_vs[k].at[pl.ds(g * G, G)]],
                rows_vs[k].at[buf], sem_g[buf]).wait()

    def compute(g, buf):
        def pt_body(p, _):
            woff = g * G + p
            wv = [plsc.load_gather(w_vs[k], [jnp.broadcast_to(woff, (16,))])
                  for k in range(K)]
            psel = jnp.broadcast_to(p, (16,))
            for c in range(C_KNOWN // 16):
                col = c * 16 + lane
                acc = wv[0] * plsc.load_gather(r0_v.at[buf], [psel, col])
                acc = acc + wv[1] * plsc.load_gather(r1_v.at[buf], [psel, col])
                acc = acc + wv[2] * plsc.load_gather(r2_v.at[buf], [psel, col])
                plsc.store_scatter(ob_v.at[buf], [psel, col], acc)
            return 0

        lax.fori_loop(0, G, pt_body, 0, unroll=False)

    def start_out(g, buf):
        pltpu.async_copy(ob_v.at[buf],
                         out_hbm.at[pl.ds(base_pt + g * G, G)], sem_o[buf])

    def wait_out(g, buf):
        pltpu.make_async_copy(ob_v.at[buf],
                              out_hbm.at[pl.ds(base_pt + g * G, G)],
                              sem_o[buf]).wait()

    start_gather(0, 0)
    start_gather(1, 1)
    for g in range(NCHUNK):
        buf = g % 2
        if g > 1:
            wait_out(g - 2, buf)
        wait_gather(g, buf)
        compute(g, buf)
        if g + 2 < NCHUNK:
            start_gather(g + 2, buf)
        start_out(g, buf)
    wait_out(NCHUNK - 2, 0)
    wait_out(NCHUNK - 1, 1)


def _interp_sc(table, idxs, ws):
    mesh = plsc.VectorSubcoreMesh(core_axis_name="c", subcore_axis_name="s")
    fn = pl.kernel(
        _interp_body,
        out_type=jax.ShapeDtypeStruct((SN, C_KNOWN), jnp.float32),
        mesh=mesh,
        compiler_params=pltpu.CompilerParams(needs_layout_passes=False),
        scratch_types=[
            pltpu.VMEM((PTS_PER_W,), jnp.int32),
            pltpu.VMEM((PTS_PER_W,), jnp.int32),
            pltpu.VMEM((PTS_PER_W,), jnp.int32),
            pltpu.VMEM((PTS_PER_W,), jnp.float32),
            pltpu.VMEM((PTS_PER_W,), jnp.float32),
            pltpu.VMEM((PTS_PER_W,), jnp.float32),
            pltpu.VMEM((2, G, C_KNOWN), jnp.float32),
            pltpu.VMEM((2, G, C_KNOWN), jnp.float32),
            pltpu.VMEM((2, G, C_KNOWN), jnp.float32),
            pltpu.VMEM((2, G, C_KNOWN), jnp.float32),
            [pltpu.SemaphoreType.DMA, pltpu.SemaphoreType.DMA],
            [pltpu.SemaphoreType.DMA, pltpu.SemaphoreType.DMA],
        ],
    )
    return fn(table, idxs[0], idxs[1], idxs[2], ws[0], ws[1], ws[2])


def _mlp1_body(xa_ref, xb_ref, wa_ref, wb_ref, b_ref, y_ref, s_ref, q_ref):
    i = pl.program_id(0)
    y = jnp.dot(xa_ref[...], wa_ref[...], preferred_element_type=jnp.float32)
    y = y + lax.dot_general(xb_ref[0], wb_ref[...],
                            (((0,), (0,)), ((), ())),
                            preferred_element_type=jnp.float32)
    y = y + b_ref[...]
    y_ref[...] = y

    @pl.when(i == 0)
    def _():
        s_ref[...] = jnp.zeros_like(s_ref)
        q_ref[...] = jnp.zeros_like(q_ref)

    s_ref[...] += jnp.sum(y, axis=0, keepdims=True)
    q_ref[...] += jnp.sum(y * y, axis=0, keepdims=True)


def _mlp1(interp, uf_full, base_b, W1aT, W1bT, b1):
    nb = SN // ROWBLK
    nbn = N // ROWBLK
    return pl.pallas_call(
        _mlp1_body,
        grid=(nb,),
        in_specs=[
            pl.BlockSpec((ROWBLK, C_KNOWN), lambda i: (i, 0)),
            pl.BlockSpec((1, C_UNKNOWN, ROWBLK),
                         lambda i: (base_b + i // nbn, 0, i % nbn)),
            pl.BlockSpec((C_KNOWN, 256), lambda i: (0, 0)),
            pl.BlockSpec((C_UNKNOWN, 256), lambda i: (0, 0)),
            pl.BlockSpec((1, 256), lambda i: (0, 0)),
        ],
        out_specs=[
            pl.BlockSpec((ROWBLK, 256), lambda i: (i, 0)),
            pl.BlockSpec((1, 256), lambda i: (0, 0)),
            pl.BlockSpec((1, 256), lambda i: (0, 0)),
        ],
        out_shape=[
            jax.ShapeDtypeStruct((SN, 256), jnp.float32),
            jax.ShapeDtypeStruct((1, 256), jnp.float32),
            jax.ShapeDtypeStruct((1, 256), jnp.float32),
        ],
    )(interp, uf_full, W1aT, W1bT, b1)


def _mlp2_body(x_ref, s1_ref, t1_ref, w_ref, b_ref, y_ref, s_ref, q_ref):
    i = pl.program_id(0)
    h = jnp.maximum(x_ref[...] * s1_ref[...] + t1_ref[...], 0.0)
    y = jnp.dot(h, w_ref[...], preferred_element_type=jnp.float32) + b_ref[...]
    y_ref[...] = y

    @pl.when(i == 0)
    def _():
        s_ref[...] = jnp.zeros_like(s_ref)
        q_ref[...] = jnp.zeros_like(q_ref)

    s_ref[...] += jnp.sum(y, axis=0, keepdims=True)
    q_ref[...] += jnp.sum(y * y, axis=0, keepdims=True)


def _mlp2(y1, s1, t1, W2T, b2):
    nb = SN // ROWBLK
    return pl.pallas_call(
        _mlp2_body,
        grid=(nb,),
        in_specs=[
            pl.BlockSpec((ROWBLK, 256), lambda i: (i, 0)),
            pl.BlockSpec((1, 256), lambda i: (0, 0)),
            pl.BlockSpec((1, 256), lambda i: (0, 0)),
            pl.BlockSpec((256, 256), lambda i: (0, 0)),
            pl.BlockSpec((1, 256), lambda i: (0, 0)),
        ],
        out_specs=[
            pl.BlockSpec((ROWBLK, 256), lambda i: (i, 0)),
            pl.BlockSpec((1, 256), lambda i: (0, 0)),
            pl.BlockSpec((1, 256), lambda i: (0, 0)),
        ],
        out_shape=[
            jax.ShapeDtypeStruct((SN, 256), jnp.float32),
            jax.ShapeDtypeStruct((1, 256), jnp.float32),
            jax.ShapeDtypeStruct((1, 256), jnp.float32),
        ],
    )(y1, s1, t1, W2T, b2)


def _final_body(y0_ref, y1_ref, y2_ref, y3_ref, s2_ref, t2_ref, out_ref):
    i = pl.program_id(0)
    y_refs = [y0_ref, y1_ref, y2_ref, y3_ref]
    nbs = SN // ROWBLK
    for s in range(SLICES):
        @pl.when(i // nbs == s)
        def _():
            h = jnp.maximum(y_refs[s][...] * s2_ref[...] + t2_ref[...], 0.0)
            out_ref[0] = h.T


def _finalize(y2s, s2, t2):
    nbs = SN // ROWBLK
    nbn = N // ROWBLK
    nb = BN // ROWBLK

    def _pinned(s):
        return pl.BlockSpec(
            (ROWBLK, 256),
            lambda i, s=s: (jnp.clip(i - s * nbs, 0, nbs - 1), 0))

    return pl.pallas_call(
        _final_body,
        grid=(nb,),
        in_specs=[_pinned(s) for s in range(SLICES)] + [
            pl.BlockSpec((1, 256), lambda i: (0, 0)),
            pl.BlockSpec((1, 256), lambda i: (0, 0)),
        ],
        out_specs=pl.BlockSpec((1, 256, ROWBLK),
                               lambda i: (i // nbn, 0, i % nbn)),
        out_shape=jax.ShapeDtypeStruct((B, 256, N), jnp.float32),
    )(*y2s, s2, t2)


def _bn_coeffs(ssum, ssq, gamma, beta):
    mean = ssum[0] / BN
    var = ssq[0] / BN - mean * mean
    s = gamma * lax.rsqrt(var + 1e-5)
    t = beta - mean * s
    return s[None, :], t[None, :]


@jax.jit
def kernel(unknown_pc, known_pc, unknow_features, known_features,
           W1, b1, gamma1, beta1, W2, b2, gamma2, beta2):
    uT = jnp.transpose(unknown_pc, (0, 2, 1))
    table = jnp.transpose(known_features, (0, 2, 1)).reshape(B * M, C_KNOWN)

    W1aT = W1[:, :C_KNOWN].T
    W1bT = W1[:, C_KNOWN:].T
    W2T = W2.T
    b1r = b1[None, :]
    b2r = b2[None, :]

    interps = []
    for s in range(SLICES):
        b0 = s * BS
        outs = _topk(uT[b0:b0 + BS], known_pc[b0:b0 + BS], b0)
        interps.append(_interp_sc(table, outs[:K], outs[K:]))

    y1s, s1sums, s1sqs = [], [], []
    for s in range(SLICES):
        y1, ssum, ssq = _mlp1(interps[s], unknow_features, s * BS,
                              W1aT, W1bT, b1r)
        y1s.append(y1)
        s1sums.append(ssum)
        s1sqs.append(ssq)

    s1, t1 = _bn_coeffs(sum(s1sums), sum(s1sqs), gamma1, beta1)

    y2s, s2sums, s2sqs = [], [], []
    for s in range(SLICES):
        y2, ssum, ssq = _mlp2(y1s[s], s1, t1, W2T, b2r)
        y2s.append(y2)
        s2sums.append(ssum)
        s2sqs.append(ssq)

    s2, t2 = _bn_coeffs(sum(s2sums), sum(s2sqs), gamma2, beta2)
    return _finalize(y2s, s2, t2)
